# Initial kernel scaffold; baseline (speedup 1.0000x reference)
import jax
import jax.numpy as jnp
from jax.experimental import pallas as pl


def kernel(x, W_in, b_in, op_W1, op_b1, op_W2, op_b2, rt_W, rt_b, W_out, b_out):
    raise NotImplementedError("write your pallas kernel here")


# speedup vs baseline: 1.7435x; 1.7435x over previous
import functools

import jax
import jax.numpy as jnp
from jax.experimental import pallas as pl
from jax.experimental.pallas import tpu as pltpu

_IN = 768
_H = 768
_OUT = 768
_T = 12
_HOPS = 4
_B = 2048
_OPS = 3


def _rng_consts():
    rkey = jax.random.key(42)
    init_t = jax.random.randint(jax.random.fold_in(rkey, 0), (_B,), 0, _T)
    ws = []
    for h in range(_HOPS):
        row = []
        for t in range(_T):
            k = jax.random.fold_in(rkey, 1000 + h * _T + t)
            row.append(jax.nn.softmax(
                jax.random.normal(k, (_OPS,), dtype=jnp.float32)))
        ws.append(jnp.stack(row))
    opw = jnp.stack(ws)
    gs = [jax.random.gumbel(jax.random.fold_in(rkey, 2000 + h),
                            (_B, _T + 1), jnp.float32) for h in range(_HOPS)]
    gum = jnp.stack(gs)
    return init_t, opw, gum


def _masked_kernel(opw_ref,
                   x_ref, w_in_ref, b_in_ref,
                   w1_ref, b1_ref, w2_ref, b2_ref,
                   rtw_ref, rtb_ref, w_out_ref, b_out_ref,
                   init_t_ref, gum_ref,
                   out_ref,
                   state, outbuf, outsel, logsel, tempers, done):
    h = pl.program_id(0)
    t = pl.program_id(1)
    o = pl.program_id(2)

    @pl.when(jnp.logical_and(h == 0, jnp.logical_and(t == 0, o == 0)))
    def _init():
        state[...] = (
            jnp.dot(x_ref[...], w_in_ref[...],
                    preferred_element_type=jnp.float32)
            + b_in_ref[...])
        tempers[...] = init_t_ref[...]
        done[...] = jnp.zeros_like(done)

    w = opw_ref[h, t, o]
    st = state[...]
    h1 = jnp.maximum(
        jnp.dot(st, w1_ref[0, 0], preferred_element_type=jnp.float32)
        + b1_ref[0, 0], 0.0)
    h2 = jnp.maximum(
        jnp.dot(h1, w2_ref[0, 0], preferred_element_type=jnp.float32)
        + b2_ref[0, 0], 0.0)

    @pl.when(o == 0)
    def _first_op():
        outbuf[...] = w * h2

    @pl.when(o != 0)
    def _acc_op():
        outbuf[...] = outbuf[...] + w * h2

    @pl.when(o == _OPS - 1)
    def _finish_temper():
        out = outbuf[...]
        nl = (jnp.dot(out, rtw_ref[0], preferred_element_type=jnp.float32)
              + rtb_ref[0])
        active = done[...] == 0
        sel = jnp.logical_and(active, tempers[...] == t)
        outsel[...] = jnp.where(sel, out, outsel[...])
        logsel[...] = jnp.where(sel, nl, logsel[...])

        @pl.when(t == _T - 1)
        def _end_hop():
            z = logsel[...] + gum_ref[0]
            m = jnp.max(z, axis=1, keepdims=True)
            ii = jax.lax.broadcasted_iota(jnp.int32, z.shape, 1)
            sampled = jnp.min(jnp.where(z >= m, ii, _T + 1),
                              axis=1, keepdims=True)
            act = done[...] == 0
            state[...] = jnp.where(act, outsel[...], state[...])
            tempers[...] = jnp.where(act, jnp.minimum(sampled, _T - 1),
                                     tempers[...])
            done[...] = jnp.where(
                jnp.logical_and(act, sampled == _T),
                jnp.ones_like(done[...]), done[...])

            @pl.when(h == _HOPS - 1)
            def _final():
                out_ref[...] = (
                    jnp.dot(state[...], w_out_ref[...],
                            preferred_element_type=jnp.float32)
                    + b_out_ref[...])


@jax.jit
def kernel(x, W_in, b_in, op_W1, op_b1, op_W2, op_b2, rt_W, rt_b, W_out,
           b_out):
    init_t, opw, gum = _rng_consts()
    init_t2 = init_t.reshape(_B, 1).astype(jnp.int32)
    b_in2 = b_in.reshape(1, _H)
    b_out2 = b_out.reshape(1, _OUT)
    op_b1r = op_b1.reshape(_T, _OPS, 1, _H)
    op_b2r = op_b2.reshape(_T, _OPS, 1, _H)
    rt_br = rt_b.reshape(_T, 1, _T + 1)

    grid = (_HOPS, _T, _OPS)
    const = lambda shape: pl.BlockSpec(shape, lambda h, t, o: (0,) * len(shape))

    out = pl.pallas_call(
        _masked_kernel,
        grid=grid,
        in_specs=[
                pl.BlockSpec(memory_space=pltpu.SMEM),
                const((_B, _IN)),
                const((_IN, _H)),
                const((1, _H)),
                pl.BlockSpec((1, 1, _H, _H), lambda h, t, o: (t, o, 0, 0)),
                pl.BlockSpec((1, 1, 1, _H), lambda h, t, o: (t, o, 0, 0)),
                pl.BlockSpec((1, 1, _H, _H), lambda h, t, o: (t, o, 0, 0)),
                pl.BlockSpec((1, 1, 1, _H), lambda h, t, o: (t, o, 0, 0)),
                pl.BlockSpec((1, _H, _T + 1), lambda h, t, o: (t, 0, 0)),
                pl.BlockSpec((1, 1, _T + 1), lambda h, t, o: (t, 0, 0)),
                const((_H, _OUT)),
                const((1, _OUT)),
                const((_B, 1)),
                pl.BlockSpec((1, _B, _T + 1), lambda h, t, o: (h, 0, 0)),
        ],
        out_specs=const((_B, _OUT)),
        scratch_shapes=[
            pltpu.VMEM((_B, _H), jnp.float32),
            pltpu.VMEM((_B, _H), jnp.float32),
            pltpu.VMEM((_B, _H), jnp.float32),
            pltpu.VMEM((_B, _T + 1), jnp.float32),
            pltpu.VMEM((_B, 1), jnp.int32),
            pltpu.VMEM((_B, 1), jnp.int32),
        ],
        out_shape=jax.ShapeDtypeStruct((_B, _OUT), jnp.float32),
        compiler_params=pltpu.CompilerParams(
            vmem_limit_bytes=100 * 1024 * 1024),
    )(opw, x, W_in, b_in2, op_W1, op_b1r, op_W2, op_b2r, rt_W, rt_br,
      W_out, b_out2, init_t2, gum)
    return out
